# baseline (speedup 1.0000x reference)
import functools

import jax
import jax.numpy as jnp
from jax.experimental import pallas as pl
from jax.experimental.pallas import tpu as pltpu

DEPTH = 10
N_NODES = 2 ** DEPTH - 1
ROUTE_TILE = 256


def _route_kernel(x_ref, nwt_ref, nb_ref, leaf_ref):
    x = x_ref[...]
    scores = jax.lax.dot_general(
        x, nwt_ref[...], (((1,), (0,)), ((), ())),
        preferred_element_type=jnp.float32,
        precision=jax.lax.Precision.HIGHEST)
    scores = scores + nb_ref[...]
    n_pad = scores.shape[1]
    cur = jnp.zeros((x.shape[0], 1), jnp.int32)
    lane = jax.lax.broadcasted_iota(jnp.int32, (x.shape[0], n_pad), 1)
    for _ in range(DEPTH):
        sel = jnp.where(lane == cur, scores, 0.0)
        s = jnp.sum(sel, axis=1, keepdims=True)
        choice = (s >= 0.0).astype(jnp.int32)
        cur = 2 * cur + 1 + choice
    leaf_ref[...] = cur - N_NODES


def _ffn_kernel(sl_ref, perm_ref, x_ref, w1_ref, b1_ref, w2_ref, b2_ref,
                out_ref):
    b = pl.program_id(0)
    l = sl_ref[b]
    xrow = x_ref[0]
    h = jnp.dot(xrow, w1_ref[0], preferred_element_type=jnp.float32)
    h = jnp.maximum(h + b1_ref[pl.ds(l, 1), :], 0.0)
    o = jnp.dot(h, w2_ref[0], preferred_element_type=jnp.float32)
    out_ref[0] = o + b2_ref[pl.ds(l, 1), :]


def kernel(x, node_weights, node_biases, w1s, b1s, w2s, b2s):
    bsz, in_w = x.shape
    n_leaves, _, leaf_w = w1s.shape
    out_w = w2s.shape[2]
    n_nodes = node_weights.shape[0]
    n_pad = n_leaves

    nwt = jnp.pad(node_weights, ((0, n_pad - n_nodes), (0, 0))).T
    nb = jnp.pad(node_biases[:, 0], (0, n_pad - n_nodes)).reshape(1, n_pad)

    leaves = pl.pallas_call(
        _route_kernel,
        grid=(bsz // ROUTE_TILE,),
        in_specs=[
            pl.BlockSpec((ROUTE_TILE, in_w), lambda i: (i, 0)),
            pl.BlockSpec((in_w, n_pad), lambda i: (0, 0)),
            pl.BlockSpec((1, n_pad), lambda i: (0, 0)),
        ],
        out_specs=pl.BlockSpec((ROUTE_TILE, 1), lambda i: (i, 0)),
        out_shape=jax.ShapeDtypeStruct((bsz, 1), jnp.int32),
    )(x, nwt, nb)[:, 0]

    order = jnp.argsort(leaves).astype(jnp.int32)
    sl = jnp.take(leaves, order).astype(jnp.int32)

    x3 = x.reshape(bsz, 1, in_w)
    grid_spec = pltpu.PrefetchScalarGridSpec(
        num_scalar_prefetch=2,
        grid=(bsz,),
        in_specs=[
            pl.BlockSpec((1, 1, in_w), lambda b, sl_r, p_r: (p_r[b], 0, 0)),
            pl.BlockSpec((1, in_w, leaf_w), lambda b, sl_r, p_r: (sl_r[b], 0, 0)),
            pl.BlockSpec((n_leaves, leaf_w), lambda b, sl_r, p_r: (0, 0)),
            pl.BlockSpec((1, leaf_w, out_w), lambda b, sl_r, p_r: (sl_r[b], 0, 0)),
            pl.BlockSpec((n_leaves, out_w), lambda b, sl_r, p_r: (0, 0)),
        ],
        out_specs=pl.BlockSpec((1, 1, out_w), lambda b, sl_r, p_r: (p_r[b], 0, 0)),
    )
    out = pl.pallas_call(
        _ffn_kernel,
        grid_spec=grid_spec,
        out_shape=jax.ShapeDtypeStruct((bsz, 1, out_w), jnp.float32),
    )(sl, order, x3, w1s, b1s, w2s, b2s)
    return out.reshape(bsz, out_w)


# speedup vs baseline: 2.6206x; 2.6206x over previous
import functools

import jax
import jax.numpy as jnp
from jax import lax
from jax.experimental import pallas as pl
from jax.experimental.pallas import tpu as pltpu
from jax.experimental.pallas import tpu_sc as plsc

DEPTH = 10
N_NODES = 2 ** DEPTH - 1
ROUTE_TILE = 256
G_LEAVES = 8
T_SLOTS = 64


def _route_kernel(x_ref, nwt_ref, nb_ref, leaf_ref, rank_ref, counts_ref):
    i = pl.program_id(0)
    x = x_ref[...]
    scores = jax.lax.dot_general(
        x, nwt_ref[...], (((1,), (0,)), ((), ())),
        preferred_element_type=jnp.float32,
        precision=jax.lax.Precision.HIGHEST)
    scores = scores + nb_ref[...]
    n_pad = scores.shape[1]
    tsz = x.shape[0]
    cur = jnp.zeros((tsz, 1), jnp.int32)
    lane = jax.lax.broadcasted_iota(jnp.int32, (tsz, n_pad), 1)
    for _ in range(DEPTH):
        sel = jnp.where(lane == cur, scores, 0.0)
        s = jnp.sum(sel, axis=1, keepdims=True)
        choice = (s >= 0.0).astype(jnp.int32)
        cur = 2 * cur + 1 + choice
    leaf = cur - N_NODES
    leaf_ref[...] = leaf

    n_groups = counts_ref.shape[1]
    group = leaf // G_LEAVES
    glane = jax.lax.broadcasted_iota(jnp.int32, (tsz, n_groups), 1)
    og = (glane == group).astype(jnp.float32)

    @pl.when(i == 0)
    def _():
        counts_ref[...] = jnp.zeros_like(counts_ref)

    run = counts_ref[...]
    r_iota = jax.lax.broadcasted_iota(jnp.int32, (tsz, tsz), 0)
    c_iota = jax.lax.broadcasted_iota(jnp.int32, (tsz, tsz), 1)
    lower = (c_iota < r_iota).astype(jnp.float32)
    prefix = jax.lax.dot_general(
        lower, og, (((1,), (0,)), ((), ())),
        preferred_element_type=jnp.float32)
    rank_in_tile = jnp.sum(prefix * og, axis=1, keepdims=True)
    rank = rank_in_tile + jnp.sum(run * og, axis=1, keepdims=True)
    rank_ref[...] = rank.astype(jnp.int32)
    counts_ref[...] = run + jnp.sum(og, axis=0, keepdims=True)


def _ffn_group_kernel(tg_ref, xs_ref, oh_ref, w1_ref, b1_ref, w2_ref, b2_ref,
                      out_ref):
    xs = xs_ref[...]
    oh = oh_ref[0]
    hs = []
    for g in range(G_LEAVES):
        h = jnp.dot(xs, w1_ref[g], preferred_element_type=jnp.float32)
        h = jnp.maximum(h + b1_ref[0, g, :][None, :], 0.0)
        hs.append(h * oh[:, g:g + 1])
    parts = [jnp.dot(oh, b2_ref[0], preferred_element_type=jnp.float32)]
    for g in range(G_LEAVES):
        parts.append(
            jnp.dot(hs[g], w2_ref[g], preferred_element_type=jnp.float32))
    while len(parts) > 1:
        nxt = [a + b for a, b in zip(parts[::2], parts[1::2])]
        if len(parts) % 2:
            nxt.append(parts[-1])
        parts = nxt
    out_ref[...] = parts[0]


@functools.lru_cache(maxsize=None)
def _make_sc_gather(n_rows, n_cols, n_idx, chunk):
    info = plsc.get_sparse_core_info()
    nw = info.num_cores * info.num_subcores
    b_per_w = n_idx // nw
    assert n_idx % nw == 0 and b_per_w % chunk == 0 and chunk % 8 == 0
    nch = b_per_w // chunk
    assert nch >= 2
    mesh = plsc.VectorSubcoreMesh(core_axis_name="c", subcore_axis_name="s")

    @functools.partial(
        pl.kernel, mesh=mesh,
        out_type=jax.ShapeDtypeStruct((n_idx, n_cols), jnp.float32),
        scratch_types=[
            pltpu.VMEM((b_per_w,), jnp.int32),
            pltpu.VMEM((chunk, n_cols), jnp.float32),
            pltpu.VMEM((chunk, n_cols), jnp.float32),
            pltpu.SemaphoreType.DMA,
            pltpu.SemaphoreType.DMA,
        ],
    )
    def sc_gather(table_hbm, idx_hbm, out_hbm, idx_v, buf0, buf1, gsem, ssem):
        wid = lax.axis_index("s") * info.num_cores + lax.axis_index("c")
        base = wid * b_per_w
        pltpu.sync_copy(idx_hbm.at[pl.ds(base, b_per_w)], idx_v)
        bufs = (buf0, buf1)
        gathers = [None] * nch
        stores = [None] * nch
        gathers[0] = pltpu.async_copy(
            table_hbm.at[idx_v.at[pl.ds(0, chunk)]], bufs[0], gsem)
        for i in range(nch):
            gathers[i].wait()
            stores[i] = pltpu.async_copy(
                bufs[i % 2], out_hbm.at[pl.ds(base + i * chunk, chunk)], ssem)
            if i + 1 < nch:
                if i >= 1:
                    stores[i - 1].wait()
                gathers[i + 1] = pltpu.async_copy(
                    table_hbm.at[idx_v.at[pl.ds((i + 1) * chunk, chunk)]],
                    bufs[(i + 1) % 2], gsem)
        stores[nch - 2].wait()
        stores[nch - 1].wait()

    return sc_gather


def kernel(x, node_weights, node_biases, w1s, b1s, w2s, b2s):
    bsz, in_w = x.shape
    n_leaves, _, leaf_w = w1s.shape
    out_w = w2s.shape[2]
    n_nodes = node_weights.shape[0]
    n_pad = n_leaves
    G, T = G_LEAVES, T_SLOTS
    n_groups = n_leaves // G

    nwt = jnp.pad(node_weights, ((0, n_pad - n_nodes), (0, 0))).T
    nb = jnp.pad(node_biases[:, 0], (0, n_pad - n_nodes)).reshape(1, n_pad)

    leaves, rank, counts = pl.pallas_call(
        _route_kernel,
        grid=(bsz // ROUTE_TILE,),
        in_specs=[
            pl.BlockSpec((ROUTE_TILE, in_w), lambda i: (i, 0)),
            pl.BlockSpec((in_w, n_pad), lambda i: (0, 0)),
            pl.BlockSpec((1, n_pad), lambda i: (0, 0)),
        ],
        out_specs=[
            pl.BlockSpec((ROUTE_TILE, 1), lambda i: (i, 0)),
            pl.BlockSpec((ROUTE_TILE, 1), lambda i: (i, 0)),
            pl.BlockSpec((1, n_groups), lambda i: (0, 0)),
        ],
        out_shape=[
            jax.ShapeDtypeStruct((bsz, 1), jnp.int32),
            jax.ShapeDtypeStruct((bsz, 1), jnp.int32),
            jax.ShapeDtypeStruct((1, n_groups), jnp.float32),
        ],
    )(x, nwt, nb)
    leaves = leaves[:, 0]
    rank = rank[:, 0]
    counts = counts[0].astype(jnp.int32)

    groups = leaves // G
    rel = leaves - groups * G
    ntiles = (counts + (T - 1)) // T
    csum = jnp.cumsum(ntiles)
    tstart = jnp.concatenate([jnp.zeros((1,), jnp.int32), csum[:-1]])
    total_tiles = csum[-1]

    nt_bound = n_groups + max(0, (bsz - n_groups)) // T
    NT = ((nt_bound * T + 255) // 256) * 256 // T
    if NT < nt_bound:
        NT = nt_bound
    t_idx = jnp.arange(NT, dtype=jnp.int32)
    t_cl = jnp.minimum(t_idx, total_tiles - 1)
    tile_group = (jnp.searchsorted(tstart, t_cl, side="right") - 1
                  ).astype(jnp.int32)

    inv = (jnp.take(tstart, groups) + rank // T) * T + rank % T
    slot_rel1 = jnp.zeros((NT * T,), jnp.int32).at[inv].set(rel + 1)
    oh = (slot_rel1[:, None] == jnp.arange(1, G + 1, dtype=jnp.int32)[None, :]
          ).astype(jnp.float32).reshape(NT, T, G)
    token_slot = jnp.zeros((NT * T,), jnp.int32).at[inv].set(
        jnp.arange(bsz, dtype=jnp.int32))

    xs = _make_sc_gather(bsz, in_w, NT * T, 32)(x, token_slot)

    b1r = b1s.reshape(n_groups, G, leaf_w)
    b2r = b2s.reshape(n_groups, G, out_w)
    grid_spec = pltpu.PrefetchScalarGridSpec(
        num_scalar_prefetch=1,
        grid=(NT,),
        in_specs=[
            pl.BlockSpec((T, in_w), lambda t, tg: (t, 0)),
            pl.BlockSpec((1, T, G), lambda t, tg: (t, 0, 0)),
            pl.BlockSpec((G, in_w, leaf_w), lambda t, tg: (tg[t], 0, 0)),
            pl.BlockSpec((1, G, leaf_w), lambda t, tg: (tg[t], 0, 0)),
            pl.BlockSpec((G, leaf_w, out_w), lambda t, tg: (tg[t], 0, 0)),
            pl.BlockSpec((1, G, out_w), lambda t, tg: (tg[t], 0, 0)),
        ],
        out_specs=pl.BlockSpec((T, out_w), lambda t, tg: (t, 0)),
    )
    out_sched = pl.pallas_call(
        _ffn_group_kernel,
        grid_spec=grid_spec,
        out_shape=jax.ShapeDtypeStruct((NT * T, out_w), jnp.float32),
    )(tile_group, xs, oh, w1s, b1r, w2s, b2r)

    out = _make_sc_gather(NT * T, out_w, bsz, 32)(out_sched, inv)
    return out


# speedup vs baseline: 3.5193x; 1.3429x over previous
import functools

import jax
import jax.numpy as jnp
from jax import lax
from jax.experimental import pallas as pl
from jax.experimental.pallas import tpu as pltpu
from jax.experimental.pallas import tpu_sc as plsc

DEPTH = 10
N_NODES = 2 ** DEPTH - 1
ROUTE_TILE = 256
G_LEAVES = 8
T_SLOTS = 64


def _route_kernel(x_ref, nwt_ref, nb_ref, leaf_ref, rank_ref, counts_ref):
    i = pl.program_id(0)
    x = x_ref[...]
    scores = jax.lax.dot_general(
        x, nwt_ref[...], (((1,), (0,)), ((), ())),
        preferred_element_type=jnp.float32,
        precision=jax.lax.Precision.HIGHEST)
    scores = scores + nb_ref[...]
    n_pad = scores.shape[1]
    tsz = x.shape[0]
    cur = jnp.zeros((tsz, 1), jnp.int32)
    lane = jax.lax.broadcasted_iota(jnp.int32, (tsz, n_pad), 1)
    for _ in range(DEPTH):
        sel = jnp.where(lane == cur, scores, 0.0)
        s = jnp.sum(sel, axis=1, keepdims=True)
        choice = (s >= 0.0).astype(jnp.int32)
        cur = 2 * cur + 1 + choice
    leaf = cur - N_NODES
    leaf_ref[...] = leaf

    n_groups = counts_ref.shape[1]
    group = leaf // G_LEAVES
    glane = jax.lax.broadcasted_iota(jnp.int32, (tsz, n_groups), 1)
    og = (glane == group).astype(jnp.float32)

    @pl.when(i == 0)
    def _():
        counts_ref[...] = jnp.zeros_like(counts_ref)

    run = counts_ref[...]
    r_iota = jax.lax.broadcasted_iota(jnp.int32, (tsz, tsz), 0)
    c_iota = jax.lax.broadcasted_iota(jnp.int32, (tsz, tsz), 1)
    lower = (c_iota < r_iota).astype(jnp.float32)
    prefix = jax.lax.dot_general(
        lower, og, (((1,), (0,)), ((), ())),
        preferred_element_type=jnp.float32)
    rank_in_tile = jnp.sum(prefix * og, axis=1, keepdims=True)
    rank = rank_in_tile + jnp.sum(run * og, axis=1, keepdims=True)
    rank_ref[...] = rank.astype(jnp.int32)
    counts_ref[...] = run + jnp.sum(og, axis=0, keepdims=True)


def _ffn_group_kernel(tg_ref, xs_ref, oh_ref, w1_ref, b1_ref, w2_ref, b2_ref,
                      out_ref):
    xs = xs_ref[...]
    oh = oh_ref[0]
    hs = []
    for g in range(G_LEAVES):
        h = jnp.dot(xs, w1_ref[g], preferred_element_type=jnp.float32)
        h = jnp.maximum(h + b1_ref[0, g, :][None, :], 0.0)
        hs.append(h * oh[:, g:g + 1])
    parts = [jnp.dot(oh, b2_ref[0], preferred_element_type=jnp.float32)]
    for g in range(G_LEAVES):
        parts.append(
            jnp.dot(hs[g], w2_ref[g], preferred_element_type=jnp.float32))
    while len(parts) > 1:
        nxt = [a + b for a, b in zip(parts[::2], parts[1::2])]
        if len(parts) % 2:
            nxt.append(parts[-1])
        parts = nxt
    out_ref[...] = parts[0]


@functools.lru_cache(maxsize=None)
def _make_sc_gather(n_rows, n_cols, n_idx, chunk):
    info = plsc.get_sparse_core_info()
    nw = info.num_cores * info.num_subcores
    b_per_w = n_idx // nw
    assert n_idx % nw == 0 and b_per_w % chunk == 0 and chunk % 8 == 0
    nch = b_per_w // chunk
    assert nch >= 2
    mesh = plsc.VectorSubcoreMesh(core_axis_name="c", subcore_axis_name="s")

    @functools.partial(
        pl.kernel, mesh=mesh,
        out_type=jax.ShapeDtypeStruct((n_idx, n_cols), jnp.float32),
        scratch_types=[
            pltpu.VMEM((b_per_w,), jnp.int32),
            pltpu.VMEM((chunk, n_cols), jnp.float32),
            pltpu.VMEM((chunk, n_cols), jnp.float32),
            pltpu.SemaphoreType.DMA,
            pltpu.SemaphoreType.DMA,
        ],
    )
    def sc_gather(table_hbm, idx_hbm, out_hbm, idx_v, buf0, buf1, gsem, ssem):
        wid = lax.axis_index("s") * info.num_cores + lax.axis_index("c")
        base = wid * b_per_w
        pltpu.sync_copy(idx_hbm.at[pl.ds(base, b_per_w)], idx_v)
        bufs = (buf0, buf1)
        gathers = [None] * nch
        stores = [None] * nch
        gathers[0] = pltpu.async_copy(
            table_hbm.at[idx_v.at[pl.ds(0, chunk)]], bufs[0], gsem)
        for i in range(nch):
            gathers[i].wait()
            stores[i] = pltpu.async_copy(
                bufs[i % 2], out_hbm.at[pl.ds(base + i * chunk, chunk)], ssem)
            if i + 1 < nch:
                if i >= 1:
                    stores[i - 1].wait()
                gathers[i + 1] = pltpu.async_copy(
                    table_hbm.at[idx_v.at[pl.ds((i + 1) * chunk, chunk)]],
                    bufs[(i + 1) % 2], gsem)
        stores[nch - 2].wait()
        stores[nch - 1].wait()

    return sc_gather


def kernel(x, node_weights, node_biases, w1s, b1s, w2s, b2s):
    bsz, in_w = x.shape
    n_leaves, _, leaf_w = w1s.shape
    out_w = w2s.shape[2]
    n_nodes = node_weights.shape[0]
    n_pad = n_leaves
    G, T = G_LEAVES, T_SLOTS
    n_groups = n_leaves // G

    nwt = jnp.pad(node_weights, ((0, n_pad - n_nodes), (0, 0))).T
    nb = jnp.pad(node_biases[:, 0], (0, n_pad - n_nodes)).reshape(1, n_pad)

    leaves, rank, counts = pl.pallas_call(
        _route_kernel,
        grid=(bsz // ROUTE_TILE,),
        in_specs=[
            pl.BlockSpec((ROUTE_TILE, in_w), lambda i: (i, 0)),
            pl.BlockSpec((in_w, n_pad), lambda i: (0, 0)),
            pl.BlockSpec((1, n_pad), lambda i: (0, 0)),
        ],
        out_specs=[
            pl.BlockSpec((ROUTE_TILE, 1), lambda i: (i, 0)),
            pl.BlockSpec((ROUTE_TILE, 1), lambda i: (i, 0)),
            pl.BlockSpec((1, n_groups), lambda i: (0, 0)),
        ],
        out_shape=[
            jax.ShapeDtypeStruct((bsz, 1), jnp.int32),
            jax.ShapeDtypeStruct((bsz, 1), jnp.int32),
            jax.ShapeDtypeStruct((1, n_groups), jnp.float32),
        ],
    )(x, nwt, nb)
    leaves = leaves[:, 0]
    rank = rank[:, 0]
    counts = counts[0].astype(jnp.int32)

    groups = leaves // G
    rel = leaves - groups * G
    ntiles = (counts + (T - 1)) // T
    csum = jnp.cumsum(ntiles)
    tstart = jnp.concatenate([jnp.zeros((1,), jnp.int32), csum[:-1]])
    total_tiles = csum[-1]

    nt_bound = n_groups + max(0, (bsz - n_groups)) // T
    NT = ((nt_bound * T + 255) // 256) * 256 // T
    if NT < nt_bound:
        NT = nt_bound
    t_idx = jnp.arange(NT, dtype=jnp.int32)
    t_cl = jnp.minimum(t_idx, total_tiles - 1)
    tile_group = (jnp.searchsorted(tstart, t_cl, side="right") - 1
                  ).astype(jnp.int32)

    inv = (jnp.take(tstart, groups) + rank // T) * T + rank % T
    slot_rel1 = jnp.zeros((NT * T,), jnp.int32).at[inv].set(rel + 1)
    oh = (slot_rel1[:, None] == jnp.arange(1, G + 1, dtype=jnp.int32)[None, :]
          ).astype(jnp.float32).reshape(NT, T, G)
    token_slot = (jnp.arange(NT * T, dtype=jnp.int32) % bsz).at[inv].set(
        jnp.arange(bsz, dtype=jnp.int32))

    xs = _make_sc_gather(bsz, in_w, NT * T, 32)(x, token_slot)

    b1r = b1s.reshape(n_groups, G, leaf_w)
    b2r = b2s.reshape(n_groups, G, out_w)
    grid_spec = pltpu.PrefetchScalarGridSpec(
        num_scalar_prefetch=1,
        grid=(NT,),
        in_specs=[
            pl.BlockSpec((T, in_w), lambda t, tg: (t, 0)),
            pl.BlockSpec((1, T, G), lambda t, tg: (t, 0, 0)),
            pl.BlockSpec((G, in_w, leaf_w), lambda t, tg: (tg[t], 0, 0)),
            pl.BlockSpec((1, G, leaf_w), lambda t, tg: (tg[t], 0, 0)),
            pl.BlockSpec((G, leaf_w, out_w), lambda t, tg: (tg[t], 0, 0)),
            pl.BlockSpec((1, G, out_w), lambda t, tg: (tg[t], 0, 0)),
        ],
        out_specs=pl.BlockSpec((T, out_w), lambda t, tg: (t, 0)),
    )
    out_sched = pl.pallas_call(
        _ffn_group_kernel,
        grid_spec=grid_spec,
        out_shape=jax.ShapeDtypeStruct((NT * T, out_w), jnp.float32),
    )(tile_group, xs, oh, w1s, b1r, w2s, b2r)

    out = _make_sc_gather(NT * T, out_w, bsz, 32)(out_sched, inv)
    return out
